# baseline (speedup 1.0000x reference)
import functools

import jax
import jax.numpy as jnp
from jax import lax
from jax.experimental import pallas as pl
from jax.experimental.pallas import tpu as pltpu
from jax.experimental.pallas import tpu_sc as plsc

_NC = 2
_NS = 16
_NW = _NC * _NS

_D = 64
_BC = 8
_GQ = 2


@functools.cache
def _make_gather(batch, fields):
    bpw = batch // _NW
    nchunk = bpw // _BC
    nidx = _BC * fields
    ipg = nidx // _GQ
    mesh = plsc.VectorSubcoreMesh(core_axis_name="c", subcore_axis_name="s")

    @functools.partial(
        pl.kernel,
        mesh=mesh,
        out_type=jax.ShapeDtypeStruct((fields, _D, batch), jnp.float32),
        scratch_types=[
            pltpu.VMEM((bpw * fields,), jnp.int32),
            pltpu.VMEM((2, nidx, _D), jnp.float32),
            pltpu.VMEM((2, fields, _D, _BC + 1), jnp.float32),
            pltpu.SemaphoreType.DMA,
            pltpu.SemaphoreType.DMA,
        ],
        compiler_params=pltpu.CompilerParams(
            use_tc_tiling_on_sc=False, needs_layout_passes=False),
    )
    def gather_kernel(idx_hbm, table_hbm, out_hbm, idx_v, rows_v, t_v,
                      gsem, wsem):
        wid = lax.axis_index("s") * _NC + lax.axis_index("c")
        base = wid * bpw * fields
        pltpu.sync_copy(idx_hbm.at[pl.ds(base, bpw * fields)], idx_v)

        lane = lax.iota(jnp.int32, 16)

        def fire_chunk_gathers(c, buf):
            for q in range(_GQ):
                pltpu.async_copy(
                    table_hbm.at[idx_v.at[pl.ds(c * nidx + q * ipg, ipg)]],
                    rows_v.at[buf, pl.ds(q * ipg, ipg)], gsem)

        def wait_chunk_gathers():
            for _ in range(_GQ):
                pltpu.make_async_copy(
                    table_hbm.at[idx_v.at[pl.ds(0, ipg)]],
                    rows_v.at[0, pl.ds(0, ipg)], gsem).wait()

        def transpose_chunk(buf):
            def fbody(f, carry):
                dst = t_v.at[buf, f]
                for k in range(_BC):
                    row = k * fields + f
                    kvec = jnp.full((16,), k, dtype=jnp.int32)
                    for d0 in range(_D // 16):
                        x = rows_v[buf, row, pl.ds(d0 * 16, 16)]
                        plsc.store_scatter(dst, [lane + d0 * 16, kvec], x)
                return carry

            lax.fori_loop(0, fields, fbody, 0)

        def fire_write(c, buf):
            pltpu.async_copy(
                t_v.at[buf, :, :, pl.ds(0, _BC)],
                out_hbm.at[:, :, pl.ds(wid * bpw + c * _BC, _BC)], wsem)

        def wait_write():
            pltpu.make_async_copy(
                t_v.at[0, :, :, pl.ds(0, _BC)],
                out_hbm.at[:, :, pl.ds(0, _BC)], wsem).wait()

        fire_chunk_gathers(0, 0)

        def body(c, carry):
            buf = lax.rem(c, 2)

            @pl.when(c + 1 < nchunk)
            def _prefetch():
                fire_chunk_gathers(c + 1, lax.rem(c + 1, 2))

            wait_chunk_gathers()

            @pl.when(c >= 2)
            def _free_tbuf():
                wait_write()

            transpose_chunk(buf)
            fire_write(c, buf)
            return carry

        lax.fori_loop(0, nchunk, body, 0)
        wait_write()
        wait_write()

    return gather_kernel


def kernel(weight, mask, input):
    b, f = input.shape
    o = _make_gather(b, f)(input.reshape(-1).astype(jnp.int32), weight)
    return jnp.transpose(o, (2, 0, 1))


# speedup vs baseline: 2.0890x; 2.0890x over previous
import functools

import jax
import jax.numpy as jnp
from jax import lax
from jax.experimental import pallas as pl
from jax.experimental.pallas import tpu as pltpu
from jax.experimental.pallas import tpu_sc as plsc

_NC = 2
_NS = 16
_NW = _NC * _NS

_D = 64
_G = 16
_PF = 32
_PD = 128


@functools.cache
def _make_gather(batch, fields):
    bpw = batch // _NW
    ngrp = bpw // _G
    mesh = plsc.VectorSubcoreMesh(core_axis_name="c", subcore_axis_name="s")

    @functools.partial(
        pl.kernel,
        mesh=mesh,
        out_type=jax.ShapeDtypeStruct((batch, _PF, _PD), jnp.float32),
        scratch_types=[
            pltpu.VMEM((bpw * _PF,), jnp.int32),
            pltpu.VMEM((2, _G, fields, _D), jnp.float32),
            pltpu.SemaphoreType.DMA,
            pltpu.SemaphoreType.DMA,
        ],
        compiler_params=pltpu.CompilerParams(use_tc_tiling_on_sc=False),
    )
    def gather_kernel(idx_hbm, table_hbm, out_hbm, idx_v, rows_v, gsem, wsem):
        wid = lax.axis_index("s") * _NC + lax.axis_index("c")
        base = wid * bpw
        pltpu.sync_copy(idx_hbm.at[pl.ds(base * _PF, bpw * _PF)], idx_v)

        def fire_group_gathers(g, buf):
            for k in range(_G):
                pltpu.async_copy(
                    table_hbm.at[idx_v.at[pl.ds((g * _G + k) * _PF,
                                                fields)]],
                    rows_v.at[buf, k], gsem)

        def wait_group_gathers():
            for _ in range(_G):
                pltpu.make_async_copy(
                    table_hbm.at[idx_v.at[pl.ds(0, fields)]],
                    rows_v.at[0, 0], gsem).wait()

        def fire_write(g, buf):
            pltpu.async_copy(
                rows_v.at[buf],
                out_hbm.at[pl.ds(base + g * _G, _G), pl.ds(0, fields),
                           pl.ds(0, _D)], wsem)

        def wait_write():
            pltpu.make_async_copy(
                rows_v.at[0],
                out_hbm.at[pl.ds(0, _G), pl.ds(0, fields), pl.ds(0, _D)],
                wsem).wait()

        fire_group_gathers(0, 0)

        def body(g, carry):
            nxt = lax.rem(g + 1, 2)

            @pl.when(g + 1 < ngrp)
            def _prefetch():
                @pl.when(g >= 1)
                def _free_buf():
                    wait_write()

                fire_group_gathers(g + 1, nxt)

            wait_group_gathers()
            fire_write(g, lax.rem(g, 2))
            return carry

        lax.fori_loop(0, ngrp, body, 0)
        wait_write()
        wait_write()

    return gather_kernel


def kernel(weight, mask, input):
    b, f = input.shape
    idx = jnp.pad(input.astype(jnp.int32), ((0, 0), (0, _PF - f)))
    o = _make_gather(b, f)(idx.reshape(-1), weight)
    return o[:, :f, :_D]
